# baseline (speedup 1.0000x reference)
import functools

import jax
import jax.numpy as jnp
from jax import lax
from jax.experimental import pallas as pl
from jax.experimental.pallas import tpu as pltpu
from jax.experimental.pallas import tpu_sc as plsc

N = 10000
E = 320000
D = 128
G = 64

NC = 2
NS = 16
NW = NC * NS
EPW = E // NW
CH = 128
NCH = 80
NCHH = NCH // 2
PADE = NCH * CH - EPW
NPAD = 10112
RPT = NPAD // NS
NDR = 79

BR = 1000
NBLK = N // BR

_MESH = dict(mesh=plsc.VectorSubcoreMesh(core_axis_name="c", subcore_axis_name="s"))


@functools.partial(
    pl.kernel,
    out_type=jax.ShapeDtypeStruct((NW, NDR, 128), jnp.float32),
    scratch_types=[
        pltpu.VMEM((NDR, 128), jnp.int32),
        pltpu.VMEM((NDR, 128), jnp.float32),
    ],
    compiler_params=pltpu.CompilerParams(needs_layout_passes=False),
    **_MESH,
)
def _sc_degree(dst_hbm, out_hbm, dst_v, cnt_v):
    c = lax.axis_index("c")
    s = lax.axis_index("s")
    wid = c * NS + s
    pltpu.sync_copy(dst_hbm.at[wid], dst_v)

    zeros = jnp.zeros((16,), jnp.float32)

    def zbody(k, carry):
        r = lax.shift_right_logical(k, 3)
        col = lax.mul(lax.bitwise_and(k, 7), 16)
        cnt_v[r, pl.ds(col, 16)] = zeros
        return carry

    lax.fori_loop(0, NDR * 8, zbody, 0)

    ones = jnp.ones((16,), jnp.float32)

    def cbody(k, carry):
        r = lax.shift_right_logical(k, 3)
        col = lax.mul(lax.bitwise_and(k, 7), 16)
        idx = dst_v[r, pl.ds(col, 16)]
        row = lax.shift_right_logical(idx, 7)
        co = lax.bitwise_and(idx, 127)
        plsc.addupdate_scatter(cnt_v, [row, co], ones)
        return carry

    lax.fori_loop(0, EPW // 16, cbody, 0)
    pltpu.sync_copy(cnt_v, out_hbm.at[wid])


@functools.partial(
    pl.kernel,
    out_type=jax.ShapeDtypeStruct((NC, NPAD, D), jnp.float32),
    scratch_types=[
        pltpu.VMEM((NCHH, CH), jnp.int32),
        pltpu.VMEM((NCHH, CH), jnp.int32),
        pltpu.VMEM((CH, D), jnp.float32),
        pltpu.VMEM((CH, D), jnp.float32),
        pltpu.VMEM_SHARED((NPAD, D), jnp.float32),
        pltpu.SemaphoreType.DMA,
        pltpu.SemaphoreType.DMA,
        pltpu.SemaphoreType.DMA,
        pltpu.SemaphoreType.DMA,
    ],
    **_MESH,
)
def _sc_scatter(src_hbm, dst_hbm, hs_hbm, out_hbm,
                src_v, dst_v, buf_a, buf_b, acc, sem_a, sem_b, sem_sa, sem_sb):
    c = lax.axis_index("c")
    s = lax.axis_index("s")
    wid = c * NS + s
    base = s * RPT

    zeros = jnp.zeros((16,), jnp.float32)

    def zb(k, carry):
        r = lax.shift_right_logical(k, 3)
        col = lax.mul(lax.bitwise_and(k, 7), 16)
        buf_a[r, pl.ds(col, 16)] = zeros
        return carry

    lax.fori_loop(0, CH * 8, zb, 0)
    for t in range(4):
        pltpu.sync_copy(buf_a, acc.at[pl.ds(base + t * CH, CH)])
    pltpu.sync_copy(buf_a.at[pl.ds(0, RPT - 4 * CH)],
                    acc.at[pl.ds(base + 4 * CH, RPT - 4 * CH)])
    plsc.subcore_barrier()

    for h in range(2):
        pltpu.sync_copy(src_hbm.at[wid].at[pl.ds(h * NCHH, NCHH)], src_v)
        pltpu.sync_copy(dst_hbm.at[wid].at[pl.ds(h * NCHH, NCHH)], dst_v)

        pltpu.async_copy(hs_hbm.at[src_v.at[0]], buf_a, sem_a)
        pltpu.async_copy(hs_hbm.at[src_v.at[1]], buf_b, sem_b)

        def body(i, carry):
            c0 = 2 * i
            pltpu.make_async_copy(hs_hbm.at[src_v.at[c0]], buf_a, sem_a).wait()
            pltpu.async_copy(buf_a, acc.at[dst_v.at[c0]], sem_sa, add=True)

            pltpu.make_async_copy(hs_hbm.at[src_v.at[c0 + 1]], buf_b, sem_b).wait()
            pltpu.async_copy(buf_b, acc.at[dst_v.at[c0 + 1]], sem_sb, add=True)

            @pl.when(c0 + 2 < NCHH)
            def _():
                pltpu.make_async_copy(buf_a, acc.at[dst_v.at[c0]], sem_sa).wait()
                pltpu.async_copy(hs_hbm.at[src_v.at[c0 + 2]], buf_a, sem_a)

            @pl.when(c0 + 3 < NCHH)
            def _():
                pltpu.make_async_copy(buf_b, acc.at[dst_v.at[c0 + 1]], sem_sb).wait()
                pltpu.async_copy(hs_hbm.at[src_v.at[c0 + 3]], buf_b, sem_b)

            return carry

        lax.fori_loop(0, NCHH // 2, body, 0)
        pltpu.make_async_copy(buf_a, acc.at[dst_v.at[0]], sem_sa).wait()
        pltpu.make_async_copy(buf_b, acc.at[dst_v.at[0]], sem_sb).wait()

    plsc.subcore_barrier()

    for t in range(4):
        bufs = (buf_a, buf_b)[t % 2]
        sems = (sem_a, sem_b)[t % 2]
        if t >= 2:
            pltpu.make_async_copy(bufs, out_hbm.at[c].at[pl.ds(base, CH)],
                                  sems).wait()
        pltpu.sync_copy(acc.at[pl.ds(base + t * CH, CH)], bufs)
        pltpu.async_copy(bufs, out_hbm.at[c].at[pl.ds(base + t * CH, CH)], sems)
    tail = RPT - 4 * CH
    pltpu.make_async_copy(buf_a, out_hbm.at[c].at[pl.ds(base, CH)], sem_a).wait()
    pltpu.sync_copy(acc.at[pl.ds(base + 4 * CH, tail)], buf_a.at[pl.ds(0, tail)])
    pltpu.sync_copy(buf_a.at[pl.ds(0, tail)],
                    out_hbm.at[c].at[pl.ds(base + 4 * CH, tail)])
    pltpu.make_async_copy(buf_b, out_hbm.at[c].at[pl.ds(base, CH)], sem_b).wait()


def _k1_body(x_ref, w_ref, cnt_ref, hs_ref, dinv_ref):
    deg = jnp.sum(cnt_ref[...], axis=1) + 1.0
    dinv = lax.rsqrt(deg)[:, None]
    h = jnp.dot(x_ref[...], w_ref[...], preferred_element_type=jnp.float32)
    hs_ref[...] = h * dinv
    dinv_ref[...] = dinv


def _kmid_body(acc_ref, hs_ref, dinv_ref, b_ref, w_ref, hs_out_ref):
    dinv = dinv_ref[...]
    z = dinv * (acc_ref[0] + acc_ref[1] + hs_ref[...]) + b_ref[...]
    a = jnp.where(z > 0, z, jnp.exp(z) - 1.0)
    h = jnp.dot(a, w_ref[...], preferred_element_type=jnp.float32)
    hs_out_ref[...] = h * dinv


def _khead_body(acc_ref, hs_ref, dinv_ref, b_ref, batch_ref, wlin_ref,
                blin_ref, out_ref, pooled):
    i = pl.program_id(0)

    @pl.when(i == 0)
    def _():
        pooled[...] = jnp.zeros_like(pooled)

    dinv = dinv_ref[...]
    z = dinv * (acc_ref[0] + acc_ref[1] + hs_ref[...]) + b_ref[...]
    a = jnp.where(z > 0, z, jnp.exp(z) - 1.0)
    seg = batch_ref[...]
    mask = (seg == lax.broadcasted_iota(jnp.int32, (1, G), 1)).astype(jnp.float32)
    pooled[...] += lax.dot_general(mask, a, (((0,), (0,)), ((), ())),
                                   preferred_element_type=jnp.float32)

    @pl.when(i == pl.num_programs(0) - 1)
    def _():
        out_ref[...] = (jnp.dot(pooled[...], wlin_ref[...],
                                preferred_element_type=jnp.float32)
                        + blin_ref[...])


_row = lambda i: (i, 0)
_all = lambda i: (0, 0)

_k1 = pl.pallas_call(
    _k1_body,
    grid=(NBLK,),
    in_specs=[
        pl.BlockSpec((BR, D), _row),
        pl.BlockSpec((D, D), _all),
        pl.BlockSpec((BR, NW), _row),
    ],
    out_specs=[
        pl.BlockSpec((BR, D), _row),
        pl.BlockSpec((BR, 1), _row),
    ],
    out_shape=[
        jax.ShapeDtypeStruct((N, D), jnp.float32),
        jax.ShapeDtypeStruct((N, 1), jnp.float32),
    ],
)

_kmid = pl.pallas_call(
    _kmid_body,
    grid=(NBLK,),
    in_specs=[
        pl.BlockSpec((NC, BR, D), lambda i: (0, i, 0)),
        pl.BlockSpec((BR, D), _row),
        pl.BlockSpec((BR, 1), _row),
        pl.BlockSpec((1, D), _all),
        pl.BlockSpec((D, D), _all),
    ],
    out_specs=pl.BlockSpec((BR, D), _row),
    out_shape=jax.ShapeDtypeStruct((N, D), jnp.float32),
)

_khead = pl.pallas_call(
    _khead_body,
    grid=(NBLK,),
    in_specs=[
        pl.BlockSpec((NC, BR, D), lambda i: (0, i, 0)),
        pl.BlockSpec((BR, D), _row),
        pl.BlockSpec((BR, 1), _row),
        pl.BlockSpec((1, D), _all),
        pl.BlockSpec((BR, 1), _row),
        pl.BlockSpec((D, 1), _all),
        pl.BlockSpec((1, 1), _all),
    ],
    out_specs=pl.BlockSpec((G, 1), _all),
    out_shape=jax.ShapeDtypeStruct((G, 1), jnp.float32),
    scratch_shapes=[pltpu.VMEM((G, D), jnp.float32)],
)


def kernel(x, edge_index, batch, W1, b1, W2, b2, W3, b3, Wlin, blin):
    srcf = edge_index[0].reshape(NW, EPW)
    dstf = edge_index[1].reshape(NW, EPW)
    pad_src = jnp.broadcast_to(
        ((jnp.arange(PADE, dtype=jnp.int32) * 41 + 7) % N)[None, :], (NW, PADE))
    pad_dst = jnp.broadcast_to(
        (N + jnp.arange(PADE, dtype=jnp.int32) % (NPAD - N))[None, :], (NW, PADE))
    src = jnp.concatenate([srcf, pad_src], axis=1).reshape(NW, NCH, CH)
    dst = jnp.concatenate([dstf, pad_dst], axis=1).reshape(NW, NCH, CH)

    dst_deg = jnp.pad(dstf, ((0, 0), (0, NDR * 128 - EPW))).reshape(NW, NDR, 128)

    batch2 = batch.reshape(N, 1)
    b1r = b1.reshape(1, D)
    b2r = b2.reshape(1, D)
    b3r = b3.reshape(1, D)
    blinr = blin.reshape(1, 1)

    counts = _sc_degree(dst_deg).reshape(NW, NDR * 128)[:, :N].T

    hs1, dinv = _k1(x, W1, counts)
    acc1 = _sc_scatter(src, dst, hs1)
    hs2 = _kmid(acc1, hs1, dinv, b1r, W2)
    acc2 = _sc_scatter(src, dst, hs2)
    hs3 = _kmid(acc2, hs2, dinv, b2r, W3)
    acc3 = _sc_scatter(src, dst, hs3)
    out = _khead(acc3, hs3, dinv, b3r, batch2, Wlin, blinr)
    return out


# speedup vs baseline: 1.2528x; 1.2528x over previous
import functools

import jax
import jax.numpy as jnp
from jax import lax
from jax.experimental import pallas as pl
from jax.experimental.pallas import tpu as pltpu
from jax.experimental.pallas import tpu_sc as plsc

N = 10000
E = 320000
D = 128
G = 64

NC = 2
NS = 16
NW = NC * NS
EPW = E // NW
CH = 128
NCH = 80
NCHH = NCH // 2
PADE = NCH * CH - EPW
NPAD = 10112
RPT = NPAD // NS
NDR = 79

BR = 1000
NBLK = N // BR

_MESH = dict(mesh=plsc.VectorSubcoreMesh(core_axis_name="c", subcore_axis_name="s"))


@functools.partial(
    pl.kernel,
    out_type=jax.ShapeDtypeStruct((NW, NDR, 128), jnp.float32),
    scratch_types=[
        pltpu.VMEM((NDR, 128), jnp.int32),
        pltpu.VMEM((NDR, 128), jnp.float32),
    ],
    compiler_params=pltpu.CompilerParams(needs_layout_passes=False),
    **_MESH,
)
def _sc_degree(dst_hbm, out_hbm, dst_v, cnt_v):
    c = lax.axis_index("c")
    s = lax.axis_index("s")
    wid = c * NS + s
    pltpu.sync_copy(dst_hbm.at[wid], dst_v)

    zeros = jnp.zeros((16,), jnp.float32)

    def zbody(k, carry):
        r = lax.shift_right_logical(k, 3)
        col = lax.mul(lax.bitwise_and(k, 7), 16)
        cnt_v[r, pl.ds(col, 16)] = zeros
        return carry

    lax.fori_loop(0, NDR * 8, zbody, 0)

    ones = jnp.ones((16,), jnp.float32)

    def cbody(k, carry):
        r = lax.shift_right_logical(k, 3)
        col = lax.mul(lax.bitwise_and(k, 7), 16)
        idx = dst_v[r, pl.ds(col, 16)]
        row = lax.shift_right_logical(idx, 7)
        co = lax.bitwise_and(idx, 127)
        plsc.addupdate_scatter(cnt_v, [row, co], ones)
        return carry

    lax.fori_loop(0, EPW // 16, cbody, 0)
    pltpu.sync_copy(cnt_v, out_hbm.at[wid])


@functools.partial(
    pl.kernel,
    out_type=jax.ShapeDtypeStruct((NC, NPAD, D), jnp.float32),
    scratch_types=[
        pltpu.VMEM((NCHH, CH), jnp.int32),
        pltpu.VMEM((NCHH, CH), jnp.int32),
        pltpu.VMEM((CH, D), jnp.float32),
        pltpu.VMEM((CH, D), jnp.float32),
        pltpu.VMEM_SHARED((NPAD, D), jnp.float32),
        pltpu.SemaphoreType.DMA,
        pltpu.SemaphoreType.DMA,
        pltpu.SemaphoreType.DMA,
        pltpu.SemaphoreType.DMA,
    ],
    **_MESH,
)
def _sc_scatter(src_hbm, dst_hbm, hs_hbm, out_hbm,
                src_v, dst_v, buf_a, buf_b, acc, sem_a, sem_b, sem_sa, sem_sb):
    c = lax.axis_index("c")
    s = lax.axis_index("s")
    wid = c * NS + s
    base = s * RPT

    zeros = jnp.zeros((16,), jnp.float32)

    def zb(k, carry):
        r = lax.shift_right_logical(k, 3)
        col = lax.mul(lax.bitwise_and(k, 7), 16)
        buf_a[r, pl.ds(col, 16)] = zeros
        return carry

    lax.fori_loop(0, CH * 8, zb, 0)
    for t in range(4):
        pltpu.sync_copy(buf_a, acc.at[pl.ds(base + t * CH, CH)])
    pltpu.sync_copy(buf_a.at[pl.ds(0, RPT - 4 * CH)],
                    acc.at[pl.ds(base + 4 * CH, RPT - 4 * CH)])
    plsc.subcore_barrier()

    for h in range(2):
        pltpu.sync_copy(src_hbm.at[wid].at[pl.ds(h * NCHH, NCHH)], src_v)
        pltpu.sync_copy(dst_hbm.at[wid].at[pl.ds(h * NCHH, NCHH)], dst_v)

        pltpu.async_copy(hs_hbm.at[src_v.at[0]], buf_a, sem_a)
        pltpu.async_copy(hs_hbm.at[src_v.at[1]], buf_b, sem_b)

        def body(i, carry):
            c0 = 2 * i
            pltpu.make_async_copy(hs_hbm.at[src_v.at[c0]], buf_a, sem_a).wait()
            pltpu.sync_copy(buf_a, acc.at[dst_v.at[c0]], add=True)

            @pl.when(c0 + 2 < NCHH)
            def _():
                pltpu.async_copy(hs_hbm.at[src_v.at[c0 + 2]], buf_a, sem_a)

            pltpu.make_async_copy(hs_hbm.at[src_v.at[c0 + 1]], buf_b, sem_b).wait()
            pltpu.sync_copy(buf_b, acc.at[dst_v.at[c0 + 1]], add=True)

            @pl.when(c0 + 3 < NCHH)
            def _():
                pltpu.async_copy(hs_hbm.at[src_v.at[c0 + 3]], buf_b, sem_b)

            return carry

        lax.fori_loop(0, NCHH // 2, body, 0)

    plsc.subcore_barrier()

    for t in range(4):
        bufs = (buf_a, buf_b)[t % 2]
        sems = (sem_a, sem_b)[t % 2]
        if t >= 2:
            pltpu.make_async_copy(bufs, out_hbm.at[c].at[pl.ds(base, CH)],
                                  sems).wait()
        pltpu.sync_copy(acc.at[pl.ds(base + t * CH, CH)], bufs)
        pltpu.async_copy(bufs, out_hbm.at[c].at[pl.ds(base + t * CH, CH)], sems)
    tail = RPT - 4 * CH
    pltpu.make_async_copy(buf_a, out_hbm.at[c].at[pl.ds(base, CH)], sem_a).wait()
    pltpu.sync_copy(acc.at[pl.ds(base + 4 * CH, tail)], buf_a.at[pl.ds(0, tail)])
    pltpu.sync_copy(buf_a.at[pl.ds(0, tail)],
                    out_hbm.at[c].at[pl.ds(base + 4 * CH, tail)])
    pltpu.make_async_copy(buf_b, out_hbm.at[c].at[pl.ds(base, CH)], sem_b).wait()


def _k1_body(x_ref, w_ref, cnt_ref, hs_ref, dinv_ref):
    deg = jnp.sum(cnt_ref[...], axis=1) + 1.0
    dinv = lax.rsqrt(deg)[:, None]
    h = jnp.dot(x_ref[...], w_ref[...], preferred_element_type=jnp.float32)
    hs_ref[...] = h * dinv
    dinv_ref[...] = dinv


def _kmid_body(acc_ref, hs_ref, dinv_ref, b_ref, w_ref, hs_out_ref):
    dinv = dinv_ref[...]
    z = dinv * (acc_ref[0] + acc_ref[1] + hs_ref[...]) + b_ref[...]
    a = jnp.where(z > 0, z, jnp.exp(z) - 1.0)
    h = jnp.dot(a, w_ref[...], preferred_element_type=jnp.float32)
    hs_out_ref[...] = h * dinv


def _khead_body(acc_ref, hs_ref, dinv_ref, b_ref, batch_ref, wlin_ref,
                blin_ref, out_ref, pooled):
    i = pl.program_id(0)

    @pl.when(i == 0)
    def _():
        pooled[...] = jnp.zeros_like(pooled)

    dinv = dinv_ref[...]
    z = dinv * (acc_ref[0] + acc_ref[1] + hs_ref[...]) + b_ref[...]
    a = jnp.where(z > 0, z, jnp.exp(z) - 1.0)
    seg = batch_ref[...]
    mask = (seg == lax.broadcasted_iota(jnp.int32, (1, G), 1)).astype(jnp.float32)
    pooled[...] += lax.dot_general(mask, a, (((0,), (0,)), ((), ())),
                                   preferred_element_type=jnp.float32)

    @pl.when(i == pl.num_programs(0) - 1)
    def _():
        out_ref[...] = (jnp.dot(pooled[...], wlin_ref[...],
                                preferred_element_type=jnp.float32)
                        + blin_ref[...])


_row = lambda i: (i, 0)
_all = lambda i: (0, 0)

_k1 = pl.pallas_call(
    _k1_body,
    grid=(NBLK,),
    in_specs=[
        pl.BlockSpec((BR, D), _row),
        pl.BlockSpec((D, D), _all),
        pl.BlockSpec((BR, NW), _row),
    ],
    out_specs=[
        pl.BlockSpec((BR, D), _row),
        pl.BlockSpec((BR, 1), _row),
    ],
    out_shape=[
        jax.ShapeDtypeStruct((N, D), jnp.float32),
        jax.ShapeDtypeStruct((N, 1), jnp.float32),
    ],
)

_kmid = pl.pallas_call(
    _kmid_body,
    grid=(NBLK,),
    in_specs=[
        pl.BlockSpec((NC, BR, D), lambda i: (0, i, 0)),
        pl.BlockSpec((BR, D), _row),
        pl.BlockSpec((BR, 1), _row),
        pl.BlockSpec((1, D), _all),
        pl.BlockSpec((D, D), _all),
    ],
    out_specs=pl.BlockSpec((BR, D), _row),
    out_shape=jax.ShapeDtypeStruct((N, D), jnp.float32),
)

_khead = pl.pallas_call(
    _khead_body,
    grid=(NBLK,),
    in_specs=[
        pl.BlockSpec((NC, BR, D), lambda i: (0, i, 0)),
        pl.BlockSpec((BR, D), _row),
        pl.BlockSpec((BR, 1), _row),
        pl.BlockSpec((1, D), _all),
        pl.BlockSpec((BR, 1), _row),
        pl.BlockSpec((D, 1), _all),
        pl.BlockSpec((1, 1), _all),
    ],
    out_specs=pl.BlockSpec((G, 1), _all),
    out_shape=jax.ShapeDtypeStruct((G, 1), jnp.float32),
    scratch_shapes=[pltpu.VMEM((G, D), jnp.float32)],
)


def kernel(x, edge_index, batch, W1, b1, W2, b2, W3, b3, Wlin, blin):
    srcf = edge_index[0].reshape(NW, EPW)
    dstf = edge_index[1].reshape(NW, EPW)
    pad_src = jnp.broadcast_to(
        ((jnp.arange(PADE, dtype=jnp.int32) * 41 + 7) % N)[None, :], (NW, PADE))
    pad_dst = jnp.broadcast_to(
        (N + jnp.arange(PADE, dtype=jnp.int32) % (NPAD - N))[None, :], (NW, PADE))
    src = jnp.concatenate([srcf, pad_src], axis=1).reshape(NW, NCH, CH)
    dst = jnp.concatenate([dstf, pad_dst], axis=1).reshape(NW, NCH, CH)

    dst_deg = jnp.pad(dstf, ((0, 0), (0, NDR * 128 - EPW))).reshape(NW, NDR, 128)

    batch2 = batch.reshape(N, 1)
    b1r = b1.reshape(1, D)
    b2r = b2.reshape(1, D)
    b3r = b3.reshape(1, D)
    blinr = blin.reshape(1, 1)

    counts = _sc_degree(dst_deg).reshape(NW, NDR * 128)[:, :N].T

    hs1, dinv = _k1(x, W1, counts)
    acc1 = _sc_scatter(src, dst, hs1)
    hs2 = _kmid(acc1, hs1, dinv, b1r, W2)
    acc2 = _sc_scatter(src, dst, hs2)
    hs3 = _kmid(acc2, hs2, dinv, b2r, W3)
    acc3 = _sc_scatter(src, dst, hs3)
    out = _khead(acc3, hs3, dinv, b3r, batch2, Wlin, blinr)
    return out
